# baseline (speedup 1.0000x reference)
import functools

import jax
import jax.numpy as jnp
from jax import lax
from jax.experimental import pallas as pl
from jax.experimental.pallas import tpu as pltpu
from jax.experimental.pallas import tpu_sc as plsc

N = 4096
NC, NS, L = 2, 16, 16
NW = NC * NS
RPW = N // NW
CR = 4
NCH = RPW // CR
NBUF = 2
UNROLL = 64
KSTEPS = (N // L) // UNROLL


def _sc_rowsum_body(data_hbm, adj_hbm, out_hbm, dbuf, abuf, sums_v, sems):
    wid = lax.axis_index("s") * NC + lax.axis_index("c")
    row0 = wid * RPW

    def start(c, b):
        r = row0 + c * CR
        pltpu.async_copy(data_hbm.at[pl.ds(r, CR), :], dbuf.at[b], sems.at[b, 0])
        pltpu.async_copy(adj_hbm.at[pl.ds(r, CR), :], abuf.at[b], sems.at[b, 1])

    def wait(b):
        pltpu.make_async_copy(data_hbm.at[pl.ds(0, CR), :], dbuf.at[b], sems.at[b, 0]).wait()
        pltpu.make_async_copy(adj_hbm.at[pl.ds(0, CR), :], abuf.at[b], sems.at[b, 1]).wait()

    for b in range(NBUF):
        start(b, b)

    def chunk_pair(i, carry):
        for b in range(NBUF):
            c = i * NBUF + b
            wait(b)
            for rr in range(CR):
                def kstep(kk, accs):
                    base = kk * UNROLL * L
                    accs = list(accs)
                    for u in range(UNROLL):
                        sl = pl.ds(base + u * L, L)
                        accs[u % 8] = accs[u % 8] + dbuf[b, rr, sl] * abuf[b, rr, sl]
                    return tuple(accs)

                accs = lax.fori_loop(
                    0, KSTEPS, kstep,
                    tuple(jnp.zeros((L,), jnp.float32) for _ in range(8)),
                )
                acc = ((accs[0] + accs[1]) + (accs[2] + accs[3])) + (
                    (accs[4] + accs[5]) + (accs[6] + accs[7]))
                sums_v[c * CR + rr, :] = acc

            @pl.when(c + NBUF < NCH)
            def _pref():
                start(c + NBUF, b)
        return carry

    lax.fori_loop(0, NCH // NBUF, chunk_pair, None)
    pltpu.sync_copy(sums_v, out_hbm.at[pl.ds(row0, RPW), :])


def _sc_rowsum(data_input, adj_matrix):
    mesh = plsc.VectorSubcoreMesh(core_axis_name="c", subcore_axis_name="s")
    f = functools.partial(
        pl.kernel,
        mesh=mesh,
        out_type=jax.ShapeDtypeStruct((N, L), jnp.float32),
        scratch_types=[
            pltpu.VMEM((NBUF, CR, N), jnp.float32),
            pltpu.VMEM((NBUF, CR, N), jnp.float32),
            pltpu.VMEM((RPW, L), jnp.float32),
            pltpu.SemaphoreType.DMA((NBUF, 2)),
        ],
    )(_sc_rowsum_body)
    return f(data_input, adj_matrix)


def _softmax_body(x_ref, out_ref):
    x = jnp.sum(x_ref[...], axis=1)
    m = jnp.max(x)
    e = jnp.exp(x - m)
    out_ref[...] = e / jnp.sum(e)


def kernel(data_input, adj_matrix):
    partials = _sc_rowsum(data_input, adj_matrix)
    alpha = pl.pallas_call(
        _softmax_body,
        out_shape=jax.ShapeDtypeStruct((N,), jnp.float32),
    )(partials)
    return alpha


# speedup vs baseline: 1.8611x; 1.8611x over previous
import functools

import jax
import jax.numpy as jnp
from jax import lax
from jax.experimental import pallas as pl
from jax.experimental.pallas import tpu as pltpu
from jax.experimental.pallas import tpu_sc as plsc

N = 4096
NC, NS, L = 2, 16, 16
NW = NC * NS

NT = 2816
NSC = N - NT
RPW = NSC // NW
CR = 4
NCH = RPW // CR
NBUF = 2
UNROLL = 64
KSTEPS = (N // L) // UNROLL

BC = 512
GRID = N // BC


def _sc_rowsum_body(data_hbm, adj_hbm, out_hbm, dbuf, abuf, sums_v, sems):
    wid = lax.axis_index("s") * NC + lax.axis_index("c")
    row0 = NT + wid * RPW

    def start(c, b):
        r = row0 + c * CR
        pltpu.async_copy(data_hbm.at[pl.ds(r, CR), :], dbuf.at[b], sems.at[b, 0])
        pltpu.async_copy(adj_hbm.at[pl.ds(r, CR), :], abuf.at[b], sems.at[b, 1])

    def wait(b):
        pltpu.make_async_copy(data_hbm.at[pl.ds(0, CR), :], dbuf.at[b], sems.at[b, 0]).wait()
        pltpu.make_async_copy(adj_hbm.at[pl.ds(0, CR), :], abuf.at[b], sems.at[b, 1]).wait()

    for b in range(NBUF):
        start(b, b)

    def chunk_pair(i, carry):
        for b in range(NBUF):
            c = i * NBUF + b
            wait(b)
            for rr in range(CR):
                def kstep(kk, accs):
                    base = kk * UNROLL * L
                    accs = list(accs)
                    for u in range(UNROLL):
                        sl = pl.ds(base + u * L, L)
                        accs[u % 8] = accs[u % 8] + dbuf[b, rr, sl] * abuf[b, rr, sl]
                    return tuple(accs)

                accs = lax.fori_loop(
                    0, KSTEPS, kstep,
                    tuple(jnp.zeros((L,), jnp.float32) for _ in range(8)),
                )
                acc = ((accs[0] + accs[1]) + (accs[2] + accs[3])) + (
                    (accs[4] + accs[5]) + (accs[6] + accs[7]))
                sums_v[c * CR + rr, :] = acc

            @pl.when(c + NBUF < NCH)
            def _pref():
                start(c + NBUF, b)
        return carry

    lax.fori_loop(0, NCH // NBUF, chunk_pair, None)
    pltpu.sync_copy(sums_v, out_hbm.at[pl.ds(wid * RPW, RPW), :])


def _sc_rowsum(data_input, adj_matrix):
    mesh = plsc.VectorSubcoreMesh(core_axis_name="c", subcore_axis_name="s")
    f = functools.partial(
        pl.kernel,
        mesh=mesh,
        out_type=jax.ShapeDtypeStruct((NSC, L), jnp.float32),
        scratch_types=[
            pltpu.VMEM((NBUF, CR, N), jnp.float32),
            pltpu.VMEM((NBUF, CR, N), jnp.float32),
            pltpu.VMEM((RPW, L), jnp.float32),
            pltpu.SemaphoreType.DMA((NBUF, 2)),
        ],
    )(_sc_rowsum_body)
    return f(data_input, adj_matrix)


def _tc_rowsum_body(data_ref, adj_ref, out_ref, acc_ref):
    i = pl.program_id(0)
    part = jnp.sum(adj_ref[...] * data_ref[...], axis=1)

    @pl.when(i == 0)
    def _init():
        acc_ref[...] = part

    @pl.when(i > 0)
    def _acc():
        acc_ref[...] += part

    @pl.when(i == GRID - 1)
    def _final():
        out_ref[...] = acc_ref[...]


def _tc_rowsum(data_input, adj_matrix):
    return pl.pallas_call(
        _tc_rowsum_body,
        grid=(GRID,),
        in_specs=[
            pl.BlockSpec((NT, BC), lambda i: (0, i)),
            pl.BlockSpec((NT, BC), lambda i: (0, i)),
        ],
        out_specs=pl.BlockSpec((NT,), lambda i: (0,)),
        out_shape=jax.ShapeDtypeStruct((NT,), jnp.float32),
        scratch_shapes=[pltpu.VMEM((NT,), jnp.float32)],
    )(data_input, adj_matrix)


def _merge_softmax_body(tc_ref, sc_ref, out_ref):
    sc_sums = jnp.sum(sc_ref[...], axis=1)
    x = jnp.concatenate([tc_ref[...], sc_sums])
    m = jnp.max(x)
    e = jnp.exp(x - m)
    out_ref[...] = e / jnp.sum(e)


def kernel(data_input, adj_matrix):
    sc_part = _sc_rowsum(data_input, adj_matrix)
    tc_sums = _tc_rowsum(data_input, adj_matrix)
    return pl.pallas_call(
        _merge_softmax_body,
        out_shape=jax.ShapeDtypeStruct((N,), jnp.float32),
    )(tc_sums, sc_part)


# speedup vs baseline: 2.4761x; 1.3304x over previous
import jax
import jax.numpy as jnp
from jax.experimental import pallas as pl
from jax.experimental.pallas import tpu as pltpu

N = 4096
BC = 256
GRID = N // BC


def _body(data_ref, adj_ref, out_ref, acc_ref):
    i = pl.program_id(0)
    part = jnp.sum(adj_ref[...] * data_ref[...], axis=1)

    @pl.when(i == 0)
    def _init():
        acc_ref[...] = part

    @pl.when(i > 0)
    def _acc():
        acc_ref[...] += part

    @pl.when(i == GRID - 1)
    def _final():
        x = acc_ref[...]
        m = jnp.max(x)
        e = jnp.exp(x - m)
        out_ref[...] = e / jnp.sum(e)


def kernel(data_input, adj_matrix):
    return pl.pallas_call(
        _body,
        grid=(GRID,),
        in_specs=[
            pl.BlockSpec((N, BC), lambda i: (0, i)),
            pl.BlockSpec((N, BC), lambda i: (0, i)),
        ],
        out_specs=pl.BlockSpec((N,), lambda i: (0,)),
        out_shape=jax.ShapeDtypeStruct((N,), jnp.float32),
        scratch_shapes=[pltpu.VMEM((N,), jnp.float32)],
    )(data_input, adj_matrix)


# speedup vs baseline: 2.9611x; 1.1959x over previous
import jax
import jax.numpy as jnp
from jax.experimental import pallas as pl
from jax.experimental.pallas import tpu as pltpu

N = 4096
BR = 512
GRID = N // BR


def _body(data_ref, adj_ref, out_ref, acc_ref):
    i = pl.program_id(0)
    acc_ref[pl.ds(i * BR, BR)] = jnp.sum(adj_ref[...] * data_ref[...], axis=1)

    @pl.when(i == GRID - 1)
    def _final():
        x = acc_ref[...]
        m = jnp.max(x)
        e = jnp.exp(x - m)
        out_ref[...] = e / jnp.sum(e)


def kernel(data_input, adj_matrix):
    return pl.pallas_call(
        _body,
        grid=(GRID,),
        in_specs=[
            pl.BlockSpec((BR, N), lambda i: (i, 0)),
            pl.BlockSpec((BR, N), lambda i: (i, 0)),
        ],
        out_specs=pl.BlockSpec((N,), lambda i: (0,)),
        out_shape=jax.ShapeDtypeStruct((N,), jnp.float32),
        scratch_shapes=[pltpu.VMEM((N,), jnp.float32)],
    )(data_input, adj_matrix)


# speedup vs baseline: 3.0126x; 1.0174x over previous
import jax
import jax.numpy as jnp
from jax.experimental import pallas as pl
from jax.experimental.pallas import tpu as pltpu

N = 4096
BR = 256
GRID = N // BR


def _body(data_ref, adj_ref, out_ref, acc_ref):
    i = pl.program_id(0)
    acc_ref[pl.ds(i * BR, BR)] = jnp.sum(adj_ref[...] * data_ref[...], axis=1)

    @pl.when(i == GRID - 1)
    def _final():
        x = acc_ref[...]
        m = jnp.max(x)
        e = jnp.exp(x - m)
        out_ref[...] = e / jnp.sum(e)


def kernel(data_input, adj_matrix):
    return pl.pallas_call(
        _body,
        grid=(GRID,),
        in_specs=[
            pl.BlockSpec((BR, N), lambda i: (i, 0)),
            pl.BlockSpec((BR, N), lambda i: (i, 0)),
        ],
        out_specs=pl.BlockSpec((N,), lambda i: (0,)),
        out_shape=jax.ShapeDtypeStruct((N,), jnp.float32),
        scratch_shapes=[pltpu.VMEM((N,), jnp.float32)],

    )(data_input, adj_matrix)
